# Initial kernel scaffold; baseline (speedup 1.0000x reference)
import jax
import jax.numpy as jnp
from jax.experimental import pallas as pl


def kernel(data, W, bias):
    raise NotImplementedError("write your pallas kernel here")


# speedup vs baseline: 1.1814x; 1.1814x over previous
import jax
import jax.numpy as jnp
from jax.experimental import pallas as pl
from jax.experimental.pallas import tpu as pltpu

_B, _I, _J, _NI, _NO = 32, 1024, 64, 16, 16
_TOPK = 16
_EPS = 1e-8

_IT = 128
_G1 = _I // _IT
_ICH = 256
_NIC = _I // _ICH


def _rb(x):
    return x.astype(jnp.bfloat16).astype(jnp.float32)


def _uhat_kernel(w2_ref, dt_ref, out_ref):
    def body(bc, carry):
        dm = [_rb(dt_ref[bc * 4 + k]) for k in range(4)]
        accs = [jnp.zeros((_J, _IT), jnp.float32) for _ in range(4)]
        for n in range(_NI):
            w = _rb(w2_ref[n, 0])
            for k in range(4):
                accs[k] = accs[k] + w * dm[k][n:n + 1, :]
        for k in range(4):
            out_ref[bc * 4 + k, 0] = accs[k]
        return carry

    jax.lax.fori_loop(0, _B // 4, body, 0)


def _squash(s):
    sq = jnp.sum(s * s, axis=-1, keepdims=True)
    return (sq / (1.0 + sq)) * s / jnp.sqrt(sq + _EPS)


def _route_kernel(ut_ref, bias_ref, out_ref, bvec_ref, c_ref):
    bias = bias_ref[...]

    cols = []
    for n in range(_NO):
        acc = None
        for ic in range(_NIC):
            sl = slice(ic * _ICH, (ic + 1) * _ICH)
            part = _rb(ut_ref[0, n, :, sl])
            acc = part if acc is None else acc + part
        cols.append(jnp.sum(acc, axis=1, keepdims=True))
    s = jnp.concatenate(cols, axis=1) * (1.0 / _J) + bias
    v = _squash(s)

    vb = _rb(v)
    for ic in range(_NIC):
        sl = slice(ic * _ICH, (ic + 1) * _ICH)
        acc = jnp.zeros((_J, _ICH), jnp.float32)
        for n in range(_NO):
            acc = acc + _rb(ut_ref[0, n, :, sl]) * vb[:, n:n + 1]
        bvec_ref[:, sl] = acc

    for it in (1, 2):
        for ic in range(_NIC):
            sl = slice(ic * _ICH, (ic + 1) * _ICH)
            t = bvec_ref[:, sl]
            gmax = jnp.max(t, axis=0, keepdims=True)
            tau = gmax
            kept = jnp.sum(jnp.where(t == gmax, 1.0, 0.0), axis=0, keepdims=True)
            w = jnp.where(t == gmax, -jnp.inf, t)
            for _step in range(_TOPK - 1):
                m = jnp.max(w, axis=0, keepdims=True)
                cnt = jnp.sum(jnp.where(w == m, 1.0, 0.0), axis=0, keepdims=True)
                active = kept < _TOPK
                tau = jnp.where(active, m, tau)
                kept = kept + jnp.where(active, cnt, 0.0)
                w = jnp.where(w == m, -jnp.inf, w)
            e = jnp.where(t >= tau, jnp.exp(t - gmax), 0.0)
            c_ref[:, sl] = e / jnp.sum(e, axis=0, keepdims=True)

        cols = []
        for n in range(_NO):
            col = None
            for ic in range(_NIC):
                sl = slice(ic * _ICH, (ic + 1) * _ICH)
                p = jnp.sum(_rb(c_ref[:, sl]) * _rb(ut_ref[0, n, :, sl]),
                            axis=1, keepdims=True)
                col = p if col is None else col + p
            cols.append(col)
        s = jnp.concatenate(cols, axis=1) + bias
        v = _squash(s)

        if it == 1:
            vb = _rb(v)
            for ic in range(_NIC):
                sl = slice(ic * _ICH, (ic + 1) * _ICH)
                acc = bvec_ref[:, sl]
                for n in range(_NO):
                    acc = acc + _rb(ut_ref[0, n, :, sl]) * vb[:, n:n + 1]
                bvec_ref[:, sl] = acc

    out_ref[0] = v


def kernel(data, W, bias):
    W2 = W.reshape(_I, _J, _NO, _NI).transpose(3, 2, 1, 0)
    dT = data.transpose(0, 2, 1)

    ut = pl.pallas_call(
        _uhat_kernel,
        grid=(_G1, _NO),
        in_specs=[
            pl.BlockSpec((_NI, 1, _J, _IT), lambda g, n: (0, n, 0, g)),
            pl.BlockSpec((_B, _NI, _IT), lambda g, n: (0, 0, g)),
        ],
        out_specs=pl.BlockSpec((_B, 1, _J, _IT), lambda g, n: (0, n, 0, g)),
        out_shape=jax.ShapeDtypeStruct((_B, _NO, _J, _I), jnp.float32),
    )(W2, dT)

    v = pl.pallas_call(
        _route_kernel,
        grid=(_B,),
        in_specs=[
            pl.BlockSpec((1, _NO, _J, _I), lambda b: (b, 0, 0, 0)),
            pl.BlockSpec((_J, _NO), lambda b: (0, 0)),
        ],
        out_specs=pl.BlockSpec((1, _J, _NO), lambda b: (b, 0, 0)),
        out_shape=jax.ShapeDtypeStruct((_B, _J, _NO), jnp.float32),
        scratch_shapes=[
            pltpu.VMEM((_J, _I), jnp.float32),
            pltpu.VMEM((_J, _I), jnp.float32),
        ],
    )(ut, bias)
    return v
